# Initial kernel scaffold; baseline (speedup 1.0000x reference)
import jax
import jax.numpy as jnp
from jax.experimental import pallas as pl


def kernel(vision, text, audio, v_mask, t_mask, a_mask, Wv1, bv1, Wv2, bv2, Wt1, bt1, Wt2, bt2, Wa1, ba1, Wa2, ba2, Wg, att_src, att_dst, edge_bias):
    raise NotImplementedError("write your pallas kernel here")


# speedup vs baseline: 424.7293x; 424.7293x over previous
import numpy as np
import jax
import jax.numpy as jnp
from jax.experimental import pallas as pl

B = 32
TV = TT = TA = 50
PER = TV + TT + TA
D = 64
H, C = 4, 16
E = B * PER * PER
K = int(E * 0.5)

_ntype = np.concatenate([np.zeros(TV), np.ones(TT), 2 * np.ones(TA)]).astype(np.int32)
_ntime = np.concatenate([np.arange(TV), np.arange(TT), np.arange(TA)]).astype(np.int32)
_trel = np.sign(_ntime[None, :] - _ntime[:, None]) + 1
_tpair = _ntype[:, None] * 3 + _ntype[None, :]
_ETYPE = (_trel * 9 + _tpair).astype(np.int32)


def _stage1(vis_ref, txt_ref, aud_ref, xm_ref,
            Wv1_ref, bv1_ref, Wv2_ref, bv2_ref,
            Wt1_ref, bt1_ref, Wt2_ref, bt2_ref,
            Wa1_ref, ba1_ref, Wa2_ref, ba2_ref,
            Wg_ref, As_ref, Ad_ref, biasD_ref,
            h_out_ref, alpha_ref, score_ref):
    f32 = jnp.float32

    def mlp(x, W1, b1, W2, b2):
        y = jnp.maximum(jnp.dot(x, W1, preferred_element_type=f32) + b1, 0.0)
        return jnp.maximum(jnp.dot(y, W2, preferred_element_type=f32) + b2, 0.0)

    v = mlp(vis_ref[0], Wv1_ref[...], bv1_ref[...], Wv2_ref[...], bv2_ref[...])
    t = mlp(txt_ref[0], Wt1_ref[...], bt1_ref[...], Wt2_ref[...], bt2_ref[...])
    a = mlp(aud_ref[0], Wa1_ref[...], ba1_ref[...], Wa2_ref[...], ba2_ref[...])
    x = jnp.concatenate([v, t, a], axis=0) * xm_ref[0]
    h = jnp.dot(x, Wg_ref[...], preferred_element_type=f32)
    h_out_ref[0] = h

    asrc = jnp.dot(h, As_ref[...], preferred_element_type=f32)
    adstT = jnp.transpose(jnp.dot(h, Ad_ref[...], preferred_element_type=f32))

    score = jnp.zeros((PER, PER), f32)
    for hh in range(H):
        eh = asrc[:, hh:hh + 1] + adstT[hh:hh + 1, :] + biasD_ref[hh]
        eh = jnp.where(eh >= 0, eh, 0.2 * eh)
        m = jnp.max(eh, axis=0, keepdims=True)
        ex = jnp.exp(eh - m)
        den = jnp.sum(ex, axis=0, keepdims=True)
        al = ex / (den + 1e-16)
        alpha_ref[0, hh] = al
        score = score + al
    score_ref[0] = score * (1.0 / H)


def _select(score_ref, thr_ref):
    bits = jax.lax.bitcast_convert_type(score_ref[...], jnp.int32)

    def body(i, prefix):
        cand = prefix | (jnp.int32(1) << (jnp.int32(30) - i))
        cnt = jnp.sum((bits >= cand).astype(jnp.int32))
        return jnp.where(cnt >= K, cand, prefix)

    prefix = jax.lax.fori_loop(0, 31, body, jnp.int32(0))
    thr_ref[...] = jnp.full((1, 1), jax.lax.bitcast_convert_type(prefix, jnp.float32))


def _stage3(alpha_ref, score_ref, h_ref, thr_ref, out_ref):
    keep = (score_ref[0] >= thr_ref[...]).astype(jnp.float32)
    h = h_ref[0]
    outs = []
    for hh in range(H):
        anum = alpha_ref[0, hh] * keep
        den2 = jnp.sum(anum, axis=0, keepdims=True)
        aln = anum / (den2 + 1e-16)
        outh = jax.lax.dot_general(aln, h[:, hh * C:(hh + 1) * C],
                                   (((0,), (0,)), ((), ())),
                                   preferred_element_type=jnp.float32)
        outs.append(outh)
    out = jnp.concatenate(outs, axis=1)
    out_ref[0] = jnp.where(out >= 0, out, 0.1 * out)


def kernel(vision, text, audio, v_mask, t_mask, a_mask,
           Wv1, bv1, Wv2, bv2, Wt1, bt1, Wt2, bt2, Wa1, ba1, Wa2, ba2,
           Wg, att_src, att_dst, edge_bias):
    f32 = jnp.float32
    xmask = jnp.concatenate([v_mask, t_mask, a_mask], axis=1).astype(f32)[:, :, None]
    eyeH = jnp.eye(H, dtype=f32)
    As = (att_src[:, :, None] * eyeH[:, None, :]).reshape(H * C, H)
    Ad = (att_dst[:, :, None] * eyeH[:, None, :]).reshape(H * C, H)
    biasD = jnp.transpose(edge_bias[jnp.asarray(_ETYPE)], (2, 0, 1))

    full = lambda shape: pl.BlockSpec(shape, lambda b: (0,) * len(shape))
    perb = lambda shape: pl.BlockSpec(shape, lambda b: (b,) + (0,) * (len(shape) - 1))

    h_nodes, alpha, score = pl.pallas_call(
        _stage1,
        grid=(B,),
        in_specs=[
            perb((1, TV, 512)), perb((1, TT, 768)), perb((1, TA, 128)),
            perb((1, PER, 1)),
            full((512, D)), full((D,)), full((D, D)), full((D,)),
            full((768, D)), full((D,)), full((D, D)), full((D,)),
            full((128, D)), full((D,)), full((D, D)), full((D,)),
            full((D, D)), full((D, H)), full((D, H)), full((H, PER, PER)),
        ],
        out_specs=[perb((1, PER, D)), perb((1, H, PER, PER)), perb((1, PER, PER))],
        out_shape=[
            jax.ShapeDtypeStruct((B, PER, D), f32),
            jax.ShapeDtypeStruct((B, H, PER, PER), f32),
            jax.ShapeDtypeStruct((B, PER, PER), f32),
        ],
    )(vision, text, audio, xmask,
      Wv1, bv1, Wv2, bv2, Wt1, bt1, Wt2, bt2, Wa1, ba1, Wa2, ba2,
      Wg, As, Ad, biasD)

    thr = pl.pallas_call(
        _select,
        out_shape=jax.ShapeDtypeStruct((1, 1), f32),
    )(score.reshape(E // 128, 128))

    out = pl.pallas_call(
        _stage3,
        grid=(B,),
        in_specs=[perb((1, H, PER, PER)), perb((1, PER, PER)), perb((1, PER, D)),
                  full((1, 1))],
        out_specs=perb((1, PER, D)),
        out_shape=jax.ShapeDtypeStruct((B, PER, D), f32),
    )(alpha, score, h_nodes, thr)

    return out.reshape(B * PER, H * C)


# speedup vs baseline: 477.2834x; 1.1237x over previous
import numpy as np
import jax
import jax.numpy as jnp
from jax.experimental import pallas as pl
from jax.experimental.pallas import tpu as pltpu

B = 32
TV = TT = TA = 50
PER = TV + TT + TA
D = 64
H, C = 4, 16
E = B * PER * PER
K = int(E * 0.5)

_ntype = np.concatenate([np.zeros(TV), np.ones(TT), 2 * np.ones(TA)]).astype(np.int32)
_ntime = np.concatenate([np.arange(TV), np.arange(TT), np.arange(TA)]).astype(np.int32)
_trel = np.sign(_ntime[None, :] - _ntime[:, None]) + 1
_tpair = _ntype[:, None] * 3 + _ntype[None, :]
_ETYPE = (_trel * 9 + _tpair).astype(np.int32)


def _fused(vis_ref, txt_ref, aud_ref, xm_ref,
           Wv1_ref, bv1_ref, Wv2_ref, bv2_ref,
           Wt1_ref, bt1_ref, Wt2_ref, bt2_ref,
           Wa1_ref, ba1_ref, Wa2_ref, ba2_ref,
           Wg_ref, As_ref, Ad_ref, biasD_ref,
           out_ref,
           score_s, h_s, asrc_s, adstT_s, m_s, den_s):
    f32 = jnp.float32
    pid = pl.program_id(0)

    @pl.when(pid < B)
    def _stage1():
        def mlp(x, W1, b1, W2, b2):
            y = jnp.maximum(jnp.dot(x, W1, preferred_element_type=f32) + b1, 0.0)
            return jnp.maximum(jnp.dot(y, W2, preferred_element_type=f32) + b2, 0.0)

        v = mlp(vis_ref[0], Wv1_ref[...], bv1_ref[...], Wv2_ref[...], bv2_ref[...])
        t = mlp(txt_ref[0], Wt1_ref[...], bt1_ref[...], Wt2_ref[...], bt2_ref[...])
        a = mlp(aud_ref[0], Wa1_ref[...], ba1_ref[...], Wa2_ref[...], ba2_ref[...])
        x = jnp.concatenate([v, t, a], axis=0) * xm_ref[0]
        h = jnp.dot(x, Wg_ref[...], preferred_element_type=f32)
        h_s[pid] = h
        asrc = jnp.dot(h, As_ref[...], preferred_element_type=f32)
        adstT = jnp.transpose(jnp.dot(h, Ad_ref[...], preferred_element_type=f32))
        asrc_s[pid] = asrc
        adstT_s[pid] = adstT

        score = jnp.zeros((PER, PER), f32)
        for hh in range(H):
            eh = asrc[:, hh:hh + 1] + adstT[hh:hh + 1, :] + biasD_ref[hh]
            eh = jnp.where(eh >= 0, eh, 0.2 * eh)
            m = jnp.max(eh, axis=0, keepdims=True)
            ex = jnp.exp(eh - m)
            den = jnp.sum(ex, axis=0, keepdims=True)
            m_s[pid, hh] = m
            den_s[pid, hh] = den
            score = score + ex / (den + 1e-16)
        score_s[pid] = score * (1.0 / H)

    @pl.when(pid == B)
    def _prune_and_aggregate():
        bits = jax.lax.bitcast_convert_type(score_s[...], jnp.int32)

        def bit_step(i, prefix):
            cand = prefix | (jnp.int32(1) << (jnp.int32(30) - i))
            cnt = jnp.sum((bits >= cand).astype(jnp.int32))
            return jnp.where(cnt >= K, cand, prefix)

        prefix = jax.lax.fori_loop(0, 31, bit_step, jnp.int32(0))
        thr = jax.lax.bitcast_convert_type(prefix, f32)

        def sample(b, carry):
            keep = (score_s[b] >= thr).astype(f32)
            asrc = asrc_s[b]
            adstT = adstT_s[b]
            h = h_s[b]
            outs = []
            for hh in range(H):
                eh = asrc[:, hh:hh + 1] + adstT[hh:hh + 1, :] + biasD_ref[hh]
                eh = jnp.where(eh >= 0, eh, 0.2 * eh)
                ex = jnp.exp(eh - m_s[b, hh])
                al = ex / (den_s[b, hh] + 1e-16)
                anum = al * keep
                den2 = jnp.sum(anum, axis=0, keepdims=True)
                aln = anum / (den2 + 1e-16)
                outs.append(jax.lax.dot_general(
                    aln, h[:, hh * C:(hh + 1) * C],
                    (((0,), (0,)), ((), ())), preferred_element_type=f32))
            o = jnp.concatenate(outs, axis=1)
            out_ref[b] = jnp.where(o >= 0, o, 0.1 * o)
            return carry

        jax.lax.fori_loop(0, B, sample, 0)


def kernel(vision, text, audio, v_mask, t_mask, a_mask,
           Wv1, bv1, Wv2, bv2, Wt1, bt1, Wt2, bt2, Wa1, ba1, Wa2, ba2,
           Wg, att_src, att_dst, edge_bias):
    f32 = jnp.float32
    xmask = jnp.concatenate([v_mask, t_mask, a_mask], axis=1).astype(f32)[:, :, None]
    eyeH = jnp.eye(H, dtype=f32)
    As = (att_src[:, :, None] * eyeH[:, None, :]).reshape(H * C, H)
    Ad = (att_dst[:, :, None] * eyeH[:, None, :]).reshape(H * C, H)
    biasD = jnp.transpose(edge_bias[jnp.asarray(_ETYPE)], (2, 0, 1))

    full = lambda shape: pl.BlockSpec(shape, lambda i: (0,) * len(shape))
    clamp = lambda shape: pl.BlockSpec(
        shape, lambda i: (jnp.minimum(i, B - 1),) + (0,) * (len(shape) - 1))

    out = pl.pallas_call(
        _fused,
        grid=(B + 1,),
        in_specs=[
            clamp((1, TV, 512)), clamp((1, TT, 768)), clamp((1, TA, 128)),
            clamp((1, PER, 1)),
            full((512, D)), full((D,)), full((D, D)), full((D,)),
            full((768, D)), full((D,)), full((D, D)), full((D,)),
            full((128, D)), full((D,)), full((D, D)), full((D,)),
            full((D, D)), full((D, H)), full((D, H)), full((H, PER, PER)),
        ],
        out_specs=pl.BlockSpec((B, PER, D), lambda i: (0, 0, 0)),
        out_shape=jax.ShapeDtypeStruct((B, PER, D), f32),
        scratch_shapes=[
            pltpu.VMEM((B, PER, PER), f32),
            pltpu.VMEM((B, PER, D), f32),
            pltpu.VMEM((B, PER, H), f32),
            pltpu.VMEM((B, H, PER), f32),
            pltpu.VMEM((B, H, 1, PER), f32),
            pltpu.VMEM((B, H, 1, PER), f32),
        ],
    )(vision, text, audio, xmask,
      Wv1, bv1, Wv2, bv2, Wt1, bt1, Wt2, bt2, Wa1, ba1, Wa2, ba2,
      Wg, As, Ad, biasD)

    return out.reshape(B * PER, H * C)


# speedup vs baseline: 998.2170x; 2.0915x over previous
import jax
import jax.numpy as jnp
from jax.experimental import pallas as pl
from jax.experimental.pallas import tpu as pltpu

B = 32
TV = TT = TA = 50
PER = TV + TT + TA
D = 64
H, C = 4, 16
E = B * PER * PER
K = int(E * 0.5)
G = 8
NSTEP = B // G


def _fused(vis_ref, txt_ref, aud_ref, xm_ref,
           Wv1_ref, bv1_ref, Wv2_ref, bv2_ref,
           Wt1_ref, bt1_ref, Wt2_ref, bt2_ref,
           Wa1_ref, ba1_ref, Wa2_ref, ba2_ref,
           Wg_ref, As_ref, Ad_ref, eb_ref,
           out_ref,
           biasD_s, score_s, h_s, ex_s):
    f32 = jnp.float32
    pid = pl.program_id(0)

    @pl.when(pid == 0)
    def _build_bias():
        ii = jax.lax.broadcasted_iota(jnp.int32, (PER, PER), 0)
        jj = jax.lax.broadcasted_iota(jnp.int32, (PER, PER), 1)
        ti = jax.lax.rem(ii, TV)
        tj = jax.lax.rem(jj, TV)
        trel = jnp.where(tj > ti, 2, jnp.where(tj < ti, 0, 1))
        etype = trel * 9 + (ii // TV) * 3 + (jj // TV)
        for hh in range(H):
            acc = jnp.zeros((PER, PER), f32)
            for n in range(27):
                acc = acc + jnp.where(etype == n, eb_ref[n, hh], 0.0)
            biasD_s[hh] = acc

    @pl.when(pid < NSTEP)
    def _stage1():
        def mlp(x, W1, b1, W2, b2):
            y = jnp.maximum(jnp.dot(x, W1, preferred_element_type=f32) + b1, 0.0)
            return jnp.maximum(jnp.dot(y, W2, preferred_element_type=f32) + b2, 0.0)

        for g in range(G):
            b = pid * G + g
            v = mlp(vis_ref[g], Wv1_ref[...], bv1_ref[...], Wv2_ref[...], bv2_ref[...])
            t = mlp(txt_ref[g], Wt1_ref[...], bt1_ref[...], Wt2_ref[...], bt2_ref[...])
            a = mlp(aud_ref[g], Wa1_ref[...], ba1_ref[...], Wa2_ref[...], ba2_ref[...])
            x = jnp.concatenate([v, t, a], axis=0) * xm_ref[g]
            h = jnp.dot(x, Wg_ref[...], preferred_element_type=f32)
            h_s[b] = h
            asrc = jnp.dot(h, As_ref[...], preferred_element_type=f32)
            adstT = jnp.transpose(jnp.dot(h, Ad_ref[...], preferred_element_type=f32))

            score = jnp.zeros((PER, PER), f32)
            for hh in range(H):
                eh = asrc[:, hh:hh + 1] + adstT[hh:hh + 1, :] + biasD_s[hh]
                eh = jnp.where(eh >= 0, eh, 0.2 * eh)
                m = jnp.max(eh, axis=0, keepdims=True)
                ex = jnp.exp(eh - m)
                ex_s[hh, b] = ex.astype(jnp.bfloat16)
                den = jnp.sum(ex, axis=0, keepdims=True)
                score = score + ex / (den + 1e-16)
            score_s[b] = score * (1.0 / H)

    @pl.when(pid == NSTEP)
    def _prune_and_aggregate():
        bits = jax.lax.bitcast_convert_type(score_s[...], jnp.int32)

        def bit_step(i, prefix):
            cand = prefix | (jnp.int32(1) << (jnp.int32(30) - i))
            cnt = jnp.sum((bits >= cand).astype(jnp.int32))
            return jnp.where(cnt >= K, cand, prefix)

        prefix = jax.lax.fori_loop(0, 31, bit_step, jnp.int32(0))
        thr = jax.lax.bitcast_convert_type(prefix, f32)

        keep = (score_s[...] >= thr).astype(f32)
        h_all = h_s[...]
        outs = []
        for hh in range(H):
            anum = ex_s[hh].astype(f32) * keep
            den2 = jnp.sum(anum, axis=1, keepdims=True)
            aln = anum / (den2 + 1e-16)
            outs.append(jax.lax.dot_general(
                aln, h_all[:, :, hh * C:(hh + 1) * C],
                (((1,), (1,)), ((0,), (0,))),
                preferred_element_type=f32))
        o = jnp.concatenate(outs, axis=2)
        out_ref[...] = jnp.where(o >= 0, o, 0.1 * o)


def kernel(vision, text, audio, v_mask, t_mask, a_mask,
           Wv1, bv1, Wv2, bv2, Wt1, bt1, Wt2, bt2, Wa1, ba1, Wa2, ba2,
           Wg, att_src, att_dst, edge_bias):
    f32 = jnp.float32
    xmask = jnp.concatenate([v_mask, t_mask, a_mask], axis=1).astype(f32)[:, :, None]
    eyeH = jnp.eye(H, dtype=f32)
    As = (att_src[:, :, None] * eyeH[:, None, :]).reshape(H * C, H)
    Ad = (att_dst[:, :, None] * eyeH[:, None, :]).reshape(H * C, H)

    full = lambda shape: pl.BlockSpec(shape, lambda i: (0,) * len(shape))
    stepb = lambda shape: pl.BlockSpec(
        shape, lambda i: (jnp.minimum(i, NSTEP - 1),) + (0,) * (len(shape) - 1))

    out = pl.pallas_call(
        _fused,
        grid=(NSTEP + 1,),
        in_specs=[
            stepb((G, TV, 512)), stepb((G, TT, 768)), stepb((G, TA, 128)),
            stepb((G, PER, 1)),
            full((512, D)), full((D,)), full((D, D)), full((D,)),
            full((768, D)), full((D,)), full((D, D)), full((D,)),
            full((128, D)), full((D,)), full((D, D)), full((D,)),
            full((D, D)), full((D, H)), full((D, H)),
            pl.BlockSpec(memory_space=pltpu.SMEM),
        ],
        out_specs=pl.BlockSpec((B, PER, D), lambda i: (0, 0, 0)),
        out_shape=jax.ShapeDtypeStruct((B, PER, D), f32),
        scratch_shapes=[
            pltpu.VMEM((H, PER, PER), f32),
            pltpu.VMEM((B, PER, PER), f32),
            pltpu.VMEM((B, PER, D), f32),
            pltpu.VMEM((H, B, PER, PER), jnp.bfloat16),
        ],
    )(vision, text, audio, xmask,
      Wv1, bv1, Wv2, bv2, Wt1, bt1, Wt2, bt2, Wa1, ba1, Wa2, ba2,
      Wg, As, Ad, edge_bias)

    return out.reshape(B * PER, H * C)


# speedup vs baseline: 1012.6396x; 1.0144x over previous
import jax
import jax.numpy as jnp
from jax.experimental import pallas as pl
from jax.experimental.pallas import tpu as pltpu

B = 32
TV = TT = TA = 50
PER = TV + TT + TA
D = 64
H, C = 4, 16
E = B * PER * PER
K = int(E * 0.5)
G = 8
NSTEP = B // G


def _fused(vis_ref, txt_ref, aud_ref, xm_ref,
           Wv1_ref, bv1_ref, Wv2_ref, bv2_ref,
           Wt1_ref, bt1_ref, Wt2_ref, bt2_ref,
           Wa1_ref, ba1_ref, Wa2_ref, ba2_ref,
           Wg_ref, As_ref, Ad_ref, eb_ref,
           out_ref,
           biasD_s, score_s, h_s, ex_s):
    f32 = jnp.float32
    pid = pl.program_id(0)

    @pl.when(pid == 0)
    def _build_bias():
        ii = jax.lax.broadcasted_iota(jnp.int32, (PER, PER), 0)
        jj = jax.lax.broadcasted_iota(jnp.int32, (PER, PER), 1)
        ti = jax.lax.rem(ii, TV)
        tj = jax.lax.rem(jj, TV)
        trel = jnp.where(tj > ti, 2, jnp.where(tj < ti, 0, 1))
        etype = trel * 9 + (ii // TV) * 3 + (jj // TV)
        for hh in range(H):
            acc = jnp.zeros((PER, PER), f32)
            for n in range(27):
                acc = acc + jnp.where(etype == n, eb_ref[n, hh], 0.0)
            biasD_s[hh] = acc

    @pl.when(pid < NSTEP)
    def _stage1():
        def mlp(x, W1, b1, W2, b2):
            y = jnp.maximum(jnp.dot(x, W1, preferred_element_type=f32) + b1, 0.0)
            return jnp.maximum(jnp.dot(y, W2, preferred_element_type=f32) + b2, 0.0)

        for g in range(G):
            b = pid * G + g
            v = mlp(vis_ref[g], Wv1_ref[...], bv1_ref[...], Wv2_ref[...], bv2_ref[...])
            t = mlp(txt_ref[g], Wt1_ref[...], bt1_ref[...], Wt2_ref[...], bt2_ref[...])
            a = mlp(aud_ref[g], Wa1_ref[...], ba1_ref[...], Wa2_ref[...], ba2_ref[...])
            x = jnp.concatenate([v, t, a], axis=0) * xm_ref[g]
            h = jnp.dot(x, Wg_ref[...], preferred_element_type=f32)
            h_s[b] = h.astype(jnp.bfloat16)
            asrc = jnp.dot(h, As_ref[...], preferred_element_type=f32)
            adstT = jnp.transpose(jnp.dot(h, Ad_ref[...], preferred_element_type=f32))

            score = jnp.zeros((PER, PER), f32)
            for hh in range(H):
                eh = asrc[:, hh:hh + 1] + adstT[hh:hh + 1, :] + biasD_s[hh]
                eh = jnp.where(eh >= 0, eh, 0.2 * eh)
                m = jnp.max(eh, axis=0, keepdims=True)
                ex = jnp.exp(eh - m)
                ex_s[hh, b] = ex.astype(jnp.bfloat16)
                den = jnp.sum(ex, axis=0, keepdims=True)
                score = score + ex / (den + 1e-16)
            score_s[b] = score * (1.0 / H)

    @pl.when(pid == NSTEP)
    def _prune_and_aggregate():
        bits = jax.lax.bitcast_convert_type(score_s[...], jnp.int32)

        def bit_step(i, prefix):
            cand = prefix | (jnp.int32(1) << (jnp.int32(30) - i))
            cnt = jnp.sum((bits >= cand).astype(jnp.int32))
            return jnp.where(cnt >= K, cand, prefix)

        prefix = jax.lax.fori_loop(0, 31, bit_step, jnp.int32(0))
        thr = jax.lax.bitcast_convert_type(prefix, f32)

        bf = jnp.bfloat16
        keep = (score_s[...] >= thr).astype(f32).astype(bf)
        h_aug = jnp.concatenate(
            [h_s[...], jnp.ones((B, PER, 1), bf)], axis=2)
        outs = []
        for hh in range(H):
            anum = ex_s[hh] * keep
            res = jax.lax.dot_general(
                anum, h_aug, (((1,), (1,)), ((0,), (0,))),
                preferred_element_type=f32)
            r = 1.0 / (res[:, :, D:D + 1] + 1e-16)
            outs.append(res[:, :, hh * C:(hh + 1) * C] * r)
        o = jnp.concatenate(outs, axis=2)
        out_ref[...] = jnp.where(o >= 0, o, 0.1 * o)


def kernel(vision, text, audio, v_mask, t_mask, a_mask,
           Wv1, bv1, Wv2, bv2, Wt1, bt1, Wt2, bt2, Wa1, ba1, Wa2, ba2,
           Wg, att_src, att_dst, edge_bias):
    f32 = jnp.float32
    xmask = jnp.concatenate([v_mask, t_mask, a_mask], axis=1).astype(f32)[:, :, None]
    eyeH = jnp.eye(H, dtype=f32)
    As = (att_src[:, :, None] * eyeH[:, None, :]).reshape(H * C, H)
    Ad = (att_dst[:, :, None] * eyeH[:, None, :]).reshape(H * C, H)

    full = lambda shape: pl.BlockSpec(shape, lambda i: (0,) * len(shape))
    stepb = lambda shape: pl.BlockSpec(
        shape, lambda i: (jnp.minimum(i, NSTEP - 1),) + (0,) * (len(shape) - 1))

    out = pl.pallas_call(
        _fused,
        grid=(NSTEP + 1,),
        in_specs=[
            stepb((G, TV, 512)), stepb((G, TT, 768)), stepb((G, TA, 128)),
            stepb((G, PER, 1)),
            full((512, D)), full((D,)), full((D, D)), full((D,)),
            full((768, D)), full((D,)), full((D, D)), full((D,)),
            full((128, D)), full((D,)), full((D, D)), full((D,)),
            full((D, D)), full((D, H)), full((D, H)),
            pl.BlockSpec(memory_space=pltpu.SMEM),
        ],
        out_specs=pl.BlockSpec((B, PER, D), lambda i: (0, 0, 0)),
        out_shape=jax.ShapeDtypeStruct((B, PER, D), f32),
        scratch_shapes=[
            pltpu.VMEM((H, PER, PER), f32),
            pltpu.VMEM((B, PER, PER), f32),
            pltpu.VMEM((B, PER, D), jnp.bfloat16),
            pltpu.VMEM((H, B, PER, PER), jnp.bfloat16),
        ],
    )(vision, text, audio, xmask,
      Wv1, bv1, Wv2, bv2, Wt1, bt1, Wt2, bt2, Wa1, ba1, Wa2, ba2,
      Wg, As, Ad, edge_bias)

    return out.reshape(B * PER, H * C)
